# Initial kernel scaffold; baseline (speedup 1.0000x reference)
import jax
import jax.numpy as jnp
from jax.experimental import pallas as pl


def kernel(input_sequence, word_embedding, position_embedding):
    raise NotImplementedError("write your pallas kernel here")


# speedup vs baseline: 4.2128x; 4.2128x over previous
import functools

import jax
import jax.numpy as jnp
from jax import lax
from jax.experimental import pallas as pl
from jax.experimental.pallas import tpu as pltpu
from jax.experimental.pallas import tpu_sc as plsc

BATCH = 4096
SEQ = 200
DIM = 64
HALF = SEQ // 2
NBUF = 4


def kernel(input_sequence, word_embedding, position_embedding):
    info = plsc.get_sparse_core_info()
    nc, ns, nl = info.num_cores, info.num_subcores, info.num_lanes
    nw = nc * ns
    rows_per_w = BATCH // nw

    idx3 = input_sequence.astype(jnp.int32).reshape(nw, 2 * rows_per_w, HALF)

    mesh = plsc.VectorSubcoreMesh(core_axis_name="c", subcore_axis_name="s")

    @functools.partial(
        pl.kernel,
        mesh=mesh,
        out_type=jax.ShapeDtypeStruct((BATCH * SEQ, DIM), jnp.float32),
        scratch_types=[
            pltpu.VMEM((2 * rows_per_w, HALF), jnp.int32),
            pltpu.VMEM((SEQ, DIM), jnp.float32),
        ]
        + [pltpu.VMEM((SEQ, DIM), jnp.float32)] * NBUF
        + [pltpu.SemaphoreType.DMA] * (2 * NBUF),
        compiler_params=pltpu.CompilerParams(use_tc_tiling_on_sc=False),
    )
    def k(idx_hbm, word_hbm, pos_hbm, out_hbm, idx_all, pos_v, *bufs):
        rbs = bufs[:NBUF]
        gsems = bufs[NBUF : 2 * NBUF]
        osems = bufs[2 * NBUF :]
        c = lax.axis_index("c")
        s = lax.axis_index("s")
        wid = s * nc + c
        base_row = wid * rows_per_w

        pltpu.sync_copy(pos_hbm, pos_v)
        pltpu.sync_copy(idx_hbm.at[wid], idx_all)

        def gather_descs(t, b):
            return [
                pltpu.make_async_copy(
                    word_hbm.at[idx_all.at[2 * t + j]],
                    rbs[b].at[pl.ds(j * HALF, HALF)],
                    gsems[b],
                )
                for j in range(2)
            ]

        def out_desc(t, b):
            return pltpu.make_async_copy(
                rbs[b],
                out_hbm.at[pl.ds((base_row + t) * SEQ, SEQ)],
                osems[b],
            )

        for p in range(NBUF - 1):
            for d in gather_descs(p, p):
                d.start()

        def quad_body(g, carry):
            for b in range(NBUF):
                t = g * NBUF + b
                for d in gather_descs(t, b):
                    d.wait()

                @plsc.parallel_loop(0, SEQ, 1, unroll=4)
                def add_body(j):
                    for kk in range(DIM // nl):
                        sl = pl.ds(kk * nl, nl)
                        plsc.addupdate(rbs[b].at[j, sl], pos_v[j, sl])

                out_desc(t, b).start()

                tb = (b + NBUF - 1) % NBUF
                tpre = t + NBUF - 1

                @pl.when(tpre < rows_per_w)
                def _():
                    @pl.when(t >= 1)
                    def _():
                        out_desc(t - 1, tb).wait()

                    for d in gather_descs(tpre, tb):
                        d.start()

            return carry

        lax.fori_loop(0, rows_per_w // NBUF, quad_body, 0)

        for b in range(NBUF):
            out_desc(0, b).wait()

    out = k(idx3, word_embedding, position_embedding)
    return out.reshape(BATCH, SEQ, DIM)


# speedup vs baseline: 4.2166x; 1.0009x over previous
import functools

import jax
import jax.numpy as jnp
from jax import lax
from jax.experimental import pallas as pl
from jax.experimental.pallas import tpu as pltpu
from jax.experimental.pallas import tpu_sc as plsc

BATCH = 4096
SEQ = 200
DIM = 64
HALF = SEQ // 2
NBUF = 4


def kernel(input_sequence, word_embedding, position_embedding):
    info = plsc.get_sparse_core_info()
    nc, ns, nl = info.num_cores, info.num_subcores, info.num_lanes
    nw = nc * ns
    rows_per_w = BATCH // nw

    idx3 = input_sequence.astype(jnp.int32).reshape(nw, 2 * rows_per_w, HALF)

    mesh = plsc.VectorSubcoreMesh(core_axis_name="c", subcore_axis_name="s")

    @functools.partial(
        pl.kernel,
        mesh=mesh,
        out_type=jax.ShapeDtypeStruct((BATCH, SEQ, DIM), jnp.float32),
        scratch_types=[
            pltpu.VMEM((2 * rows_per_w, HALF), jnp.int32),
            pltpu.VMEM((SEQ, DIM), jnp.float32),
        ]
        + [pltpu.VMEM((SEQ, DIM), jnp.float32)] * NBUF
        + [pltpu.SemaphoreType.DMA] * (2 * NBUF),
        compiler_params=pltpu.CompilerParams(use_tc_tiling_on_sc=False),
    )
    def k(idx_hbm, word_hbm, pos_hbm, out_hbm, idx_all, pos_v, *bufs):
        rbs = bufs[:NBUF]
        gsems = bufs[NBUF : 2 * NBUF]
        osems = bufs[2 * NBUF :]
        c = lax.axis_index("c")
        s = lax.axis_index("s")
        wid = s * nc + c
        base_row = wid * rows_per_w

        pltpu.sync_copy(pos_hbm, pos_v)
        pltpu.sync_copy(idx_hbm.at[wid], idx_all)

        def gather_descs(t, b):
            return [
                pltpu.make_async_copy(
                    word_hbm.at[idx_all.at[2 * t + j]],
                    rbs[b].at[pl.ds(j * HALF, HALF)],
                    gsems[b],
                )
                for j in range(2)
            ]

        def out_desc(t, b):
            return pltpu.make_async_copy(
                rbs[b],
                out_hbm.at[base_row + t],
                osems[b],
            )

        for p in range(NBUF - 1):
            for d in gather_descs(p, p):
                d.start()

        def quad_body(g, carry):
            for b in range(NBUF):
                t = g * NBUF + b
                for d in gather_descs(t, b):
                    d.wait()

                @plsc.parallel_loop(0, SEQ, 1, unroll=4)
                def add_body(j):
                    for kk in range(DIM // nl):
                        sl = pl.ds(kk * nl, nl)
                        plsc.addupdate(rbs[b].at[j, sl], pos_v[j, sl])

                out_desc(t, b).start()

                tb = (b + NBUF - 1) % NBUF
                tpre = t + NBUF - 1

                @pl.when(tpre < rows_per_w)
                def _():
                    @pl.when(t >= 1)
                    def _():
                        out_desc(t - 1, tb).wait()

                    for d in gather_descs(tpre, tb):
                        d.start()

            return carry

        lax.fori_loop(0, rows_per_w // NBUF, quad_body, 0)

        for b in range(NBUF):
            out_desc(0, b).wait()

    return k(idx3, word_embedding, position_embedding)


# speedup vs baseline: 7.4041x; 1.7559x over previous
import functools

import jax
import jax.numpy as jnp
from jax import lax
from jax.experimental import pallas as pl
from jax.experimental.pallas import tpu as pltpu
from jax.experimental.pallas import tpu_sc as plsc

BATCH = 4096
SEQ = 200
DIM = 64
HALF = SEQ // 2
NBUF = 4


def kernel(input_sequence, word_embedding, position_embedding):
    info = plsc.get_sparse_core_info()
    nc, ns, nl = info.num_cores, info.num_subcores, info.num_lanes
    nw = nc * ns
    rows_per_w = BATCH // nw

    idx3 = input_sequence.astype(jnp.int32).reshape(nw, 2 * rows_per_w, HALF)

    mesh = plsc.VectorSubcoreMesh(core_axis_name="c", subcore_axis_name="s")

    @functools.partial(
        pl.kernel,
        mesh=mesh,
        out_type=jax.ShapeDtypeStruct((BATCH, SEQ, 128), jnp.float32),
        scratch_types=[
            pltpu.VMEM((2 * rows_per_w, HALF), jnp.int32),
            pltpu.VMEM((SEQ, DIM), jnp.float32),
        ]
        + [pltpu.VMEM((SEQ, DIM), jnp.float32)] * NBUF
        + [pltpu.SemaphoreType.DMA] * (2 * NBUF),
        compiler_params=pltpu.CompilerParams(use_tc_tiling_on_sc=False),
    )
    def k(idx_hbm, word_hbm, pos_hbm, out_hbm, idx_all, pos_v, *bufs):
        rbs = bufs[:NBUF]
        gsems = bufs[NBUF : 2 * NBUF]
        osems = bufs[2 * NBUF :]
        c = lax.axis_index("c")
        s = lax.axis_index("s")
        wid = s * nc + c
        base_row = wid * rows_per_w

        pltpu.sync_copy(pos_hbm, pos_v)
        pltpu.sync_copy(idx_hbm.at[wid], idx_all)

        def gather_descs(t, b):
            return [
                pltpu.make_async_copy(
                    word_hbm.at[idx_all.at[2 * t + j]],
                    rbs[b].at[pl.ds(j * HALF, HALF)],
                    gsems[b],
                )
                for j in range(2)
            ]

        def out_desc(t, b):
            return pltpu.make_async_copy(
                rbs[b],
                out_hbm.at[base_row + t, :, pl.ds(0, DIM)],
                osems[b],
            )

        for p in range(NBUF - 1):
            for d in gather_descs(p, p):
                d.start()

        def quad_body(g, carry):
            for b in range(NBUF):
                t = g * NBUF + b
                for d in gather_descs(t, b):
                    d.wait()

                @plsc.parallel_loop(0, SEQ, 1, unroll=4)
                def add_body(j):
                    for kk in range(DIM // nl):
                        sl = pl.ds(kk * nl, nl)
                        plsc.addupdate(rbs[b].at[j, sl], pos_v[j, sl])

                out_desc(t, b).start()

                tb = (b + NBUF - 1) % NBUF
                tpre = t + NBUF - 1

                @pl.when(tpre < rows_per_w)
                def _():
                    @pl.when(t >= 1)
                    def _():
                        out_desc(t - 1, tb).wait()

                    for d in gather_descs(tpre, tb):
                        d.start()

            return carry

        lax.fori_loop(0, rows_per_w // NBUF, quad_body, 0)

        for b in range(NBUF):
            out_desc(0, b).wait()

    out = k(idx3, word_embedding, position_embedding)
    return jax.lax.slice(out, (0, 0, 0), (BATCH, SEQ, DIM))
